# Initial kernel scaffold; baseline (speedup 1.0000x reference)
import jax
import jax.numpy as jnp
from jax.experimental import pallas as pl


def kernel(z_e_1, z_e_2, E):
    raise NotImplementedError("write your pallas kernel here")


# speedup vs baseline: 1.0180x; 1.0180x over previous
import jax
import jax.numpy as jnp
from jax.experimental import pallas as pl

BETA = 0.25


def _vq_body(z1_ref, z2_ref, e_ref, zq_ref, idx_ref, dp_ref):
    Ew = e_ref[...]
    e_sq = jnp.sum(Ew * Ew, axis=1)
    d = None
    for z_ref in (z1_ref, z2_ref):
        zf = z_ref[0].T
        m = jax.lax.dot_general(
            zf, Ew, (((1,), (1,)), ((), ())),
            preferred_element_type=jnp.float32)
        zsq = jnp.sum(zf * zf, axis=1, keepdims=True)
        dist = (zsq + e_sq[None, :]) - 2.0 * m
        d = dist if d is None else d + dist
    P, K = d.shape
    dmin = jnp.min(d, axis=1)
    iota = jax.lax.broadcasted_iota(jnp.int32, (P, K), 1)
    idx = jnp.min(jnp.where(d == dmin[:, None], iota, K), axis=1)
    oh = (iota == idx[:, None]).astype(jnp.float32)
    zq = jax.lax.dot_general(
        Ew, oh, (((0,), (1,)), ((), ())),
        preferred_element_type=jnp.float32,
        precision=jax.lax.Precision.HIGHEST)
    zq_ref[0] = zq
    idx_ref[0, 0] = idx
    dp_ref[0, 0] = jnp.full((128,), jnp.sum(dmin) * 0.5, jnp.float32)


def kernel(z_e_1, z_e_2, E):
    B, C, H, W = z_e_1.shape
    P = H * W
    K = E.shape[0]
    z1 = z_e_1.reshape(B, C, P)
    z2 = z_e_2.reshape(B, C, P)

    zq, idx, dparts = pl.pallas_call(
        _vq_body,
        grid=(B,),
        in_specs=[
            pl.BlockSpec((1, C, P), lambda b: (b, 0, 0)),
            pl.BlockSpec((1, C, P), lambda b: (b, 0, 0)),
            pl.BlockSpec((K, C), lambda b: (0, 0)),
        ],
        out_specs=[
            pl.BlockSpec((1, C, P), lambda b: (b, 0, 0)),
            pl.BlockSpec((1, 1, P), lambda b: (b, 0, 0)),
            pl.BlockSpec((1, 1, 128), lambda b: (b, 0, 0)),
        ],
        out_shape=[
            jax.ShapeDtypeStruct((B, C, P), jnp.float32),
            jax.ShapeDtypeStruct((B, 1, P), jnp.int32),
            jax.ShapeDtypeStruct((B, 1, 128), jnp.float32),
        ],
    )(z1, z2, E)

    z_q = zq.reshape(B, C, H, W)
    indices = idx.reshape(B * P)
    n_el = float(B * C * H * W)
    codebook_loss = jnp.sum(dparts[:, 0, 0]) / n_el
    commitment_loss = codebook_loss
    vq_loss = codebook_loss + BETA * commitment_loss
    return (z_q, codebook_loss, commitment_loss, vq_loss, indices)


# speedup vs baseline: 1.7796x; 1.7481x over previous
import jax
import jax.numpy as jnp
from jax.experimental import pallas as pl

BETA = 0.25


def _vq_body(z1_ref, z2_ref, e_ref, zq_ref, idx_ref, dp_ref):
    Ew = e_ref[...]
    e_sq = jnp.sum(Ew * Ew, axis=1)
    d = None
    for z_ref in (z1_ref, z2_ref):
        zb = z_ref[0]
        m = jax.lax.dot_general(
            Ew, zb, (((1,), (0,)), ((), ())),
            preferred_element_type=jnp.float32)
        zsq = jnp.sum(zb.T * zb.T, axis=1)
        dist = (e_sq[:, None] + zsq[None, :]) - 2.0 * m
        d = dist if d is None else d + dist
    K, P = d.shape
    dmin = jnp.min(d, axis=0)
    iota = jax.lax.broadcasted_iota(jnp.int32, (K, P), 0)
    idx = jnp.min(jnp.where(d == dmin[None, :], iota, K), axis=0)
    oh = (iota == idx[None, :]).astype(jnp.bfloat16)
    zq = jax.lax.dot_general(
        Ew.astype(jnp.bfloat16), oh, (((0,), (0,)), ((), ())),
        preferred_element_type=jnp.float32)
    zq_ref[0] = zq
    idx_ref[0, 0] = idx
    dp_ref[0, 0] = jnp.full((128,), jnp.sum(dmin) * 0.5, jnp.float32)


def kernel(z_e_1, z_e_2, E):
    B, C, H, W = z_e_1.shape
    P = H * W
    K = E.shape[0]
    z1 = z_e_1.reshape(B, C, P)
    z2 = z_e_2.reshape(B, C, P)

    zq, idx, dparts = pl.pallas_call(
        _vq_body,
        grid=(B,),
        in_specs=[
            pl.BlockSpec((1, C, P), lambda b: (b, 0, 0)),
            pl.BlockSpec((1, C, P), lambda b: (b, 0, 0)),
            pl.BlockSpec((K, C), lambda b: (0, 0)),
        ],
        out_specs=[
            pl.BlockSpec((1, C, P), lambda b: (b, 0, 0)),
            pl.BlockSpec((1, 1, P), lambda b: (b, 0, 0)),
            pl.BlockSpec((1, 1, 128), lambda b: (b, 0, 0)),
        ],
        out_shape=[
            jax.ShapeDtypeStruct((B, C, P), jnp.float32),
            jax.ShapeDtypeStruct((B, 1, P), jnp.int32),
            jax.ShapeDtypeStruct((B, 1, 128), jnp.float32),
        ],
    )(z1, z2, E)

    z_q = zq.reshape(B, C, H, W)
    indices = idx.reshape(B * P)
    n_el = float(B * C * H * W)
    codebook_loss = jnp.sum(dparts[:, 0, 0]) / n_el
    commitment_loss = codebook_loss
    vq_loss = codebook_loss + BETA * commitment_loss
    return (z_q, codebook_loss, commitment_loss, vq_loss, indices)
